# baseline (speedup 1.0000x reference)
import functools
import math

import jax
import jax.numpy as jnp
from jax import lax
from jax.experimental import pallas as pl
from jax.experimental.pallas import tpu as pltpu
from jax.experimental.pallas import tpu_sc as plsc

_INFO = plsc.get_sparse_core_info()
_NC = _INFO.num_cores
_NS = _INFO.num_subcores
_NW = _NC * _NS
_LANES = _INFO.num_lanes

_CHUNK = 128


def _make_gather(n_chunks: int, dim: int):
    mesh = plsc.VectorSubcoreMesh(core_axis_name="c", subcore_axis_name="s")

    @functools.partial(
        pl.kernel,
        mesh=mesh,
        out_type=jax.ShapeDtypeStruct((_NW, n_chunks, _CHUNK, dim), jnp.float32),
        scratch_types=[
            pltpu.VMEM((n_chunks, _CHUNK), jnp.int32),
            pltpu.VMEM((_CHUNK, dim), jnp.float32),
            pltpu.SemaphoreType.DMA,
        ],
        compiler_params=pltpu.CompilerParams(use_tc_tiling_on_sc=False),
    )
    def k(idx_hbm, table_hbm, out_hbm, idx_v, rows_v, sem):
        wid = lax.axis_index("s") * _NC + lax.axis_index("c")
        pltpu.sync_copy(idx_hbm.at[wid], idx_v)

        def chunk_body(g, carry):
            pltpu.async_copy(table_hbm.at[idx_v.at[g]], rows_v, sem).wait()

            def row_body(i, c2):
                for j in range(dim // _LANES):
                    sl = pl.ds(j * _LANES, _LANES)
                    rows_v[i, sl] = rows_v[i, sl] * 8.0
                return c2

            lax.fori_loop(0, _CHUNK, row_body, 0)
            pltpu.sync_copy(rows_v, out_hbm.at[wid, g])
            return carry

        lax.fori_loop(0, n_chunks, chunk_body, 0)

    return k


def kernel(indices, W):
    batch, hist = indices.shape
    vocab, dim = W.shape
    total = batch * hist
    assert total % (_NW * _CHUNK) == 0 and dim % _LANES == 0
    n_chunks = total // (_NW * _CHUNK)

    idx = indices.reshape(_NW, n_chunks, _CHUNK).astype(jnp.int32)
    out = _make_gather(n_chunks, dim)(idx, W)
    return out.reshape(batch, hist, dim)


# speedup vs baseline: 1.2019x; 1.2019x over previous
import functools
import math

import jax
import jax.numpy as jnp
from jax import lax
from jax.experimental import pallas as pl
from jax.experimental.pallas import tpu as pltpu
from jax.experimental.pallas import tpu_sc as plsc

_INFO = plsc.get_sparse_core_info()
_NC = _INFO.num_cores
_NS = _INFO.num_subcores
_NW = _NC * _NS
_LANES = _INFO.num_lanes

_CHUNK = 128
_GRP = 4
_ROW_UNROLL = 8


def _make_gather(n_chunks: int, dim: int):
    mesh = plsc.VectorSubcoreMesh(core_axis_name="c", subcore_axis_name="s")
    n_groups = n_chunks // _GRP
    assert n_chunks % _GRP == 0 and n_groups % 2 == 0 and n_groups >= 4
    n_pairs = n_groups // 2

    scratch = [pltpu.VMEM((n_chunks, _CHUNK), jnp.int32)]
    scratch += [pltpu.VMEM((_CHUNK, dim), jnp.float32) for _ in range(2 * _GRP)]
    scratch += [pltpu.SemaphoreType.DMA for _ in range(4)]

    @functools.partial(
        pl.kernel,
        mesh=mesh,
        out_type=jax.ShapeDtypeStruct((_NW, n_chunks, _CHUNK, dim), jnp.float32),
        scratch_types=scratch,
        compiler_params=pltpu.CompilerParams(use_tc_tiling_on_sc=False),
    )
    def k(idx_hbm, table_hbm, out_hbm, idx_v, *rest):
        bufs = (rest[:_GRP], rest[_GRP:2 * _GRP])
        sem_g = rest[2 * _GRP:2 * _GRP + 2]
        sem_s = rest[2 * _GRP + 2:2 * _GRP + 4]

        wid = lax.axis_index("s") * _NC + lax.axis_index("c")
        pltpu.sync_copy(idx_hbm.at[wid], idx_v)

        def issue_gathers(grp, bank):
            for j in range(_GRP):
                pltpu.async_copy(
                    table_hbm.at[idx_v.at[grp * _GRP + j]],
                    bufs[bank][j], sem_g[bank])

        def drain_gathers(grp, bank):
            for j in range(_GRP):
                pltpu.make_async_copy(
                    table_hbm.at[idx_v.at[grp * _GRP + j]],
                    bufs[bank][j], sem_g[bank]).wait()

        def scale(bank, j):
            buf = bufs[bank][j]

            def body(i, c):
                base = i * _ROW_UNROLL
                for r in range(_ROW_UNROLL):
                    for col in range(dim // _LANES):
                        sl = pl.ds(col * _LANES, _LANES)
                        buf[base + r, sl] = buf[base + r, sl] * 8.0
                return c

            lax.fori_loop(0, _CHUNK // _ROW_UNROLL, body, 0)

        def issue_stores(grp, bank):
            for j in range(_GRP):
                scale(bank, j)
                pltpu.async_copy(
                    bufs[bank][j], out_hbm.at[wid, grp * _GRP + j],
                    sem_s[bank])

        def drain_stores(grp, bank):
            for j in range(_GRP):
                pltpu.make_async_copy(
                    bufs[bank][j], out_hbm.at[wid, grp * _GRP + j],
                    sem_s[bank]).wait()

        def visit(grp, bank, next_grp):
            drain_gathers(grp, bank)
            issue_stores(grp, bank)
            if next_grp is not None:
                drain_stores(grp, bank)
                issue_gathers(next_grp, bank)

        issue_gathers(0, 0)
        issue_gathers(1, 1)

        def pair(p, c):
            g0 = 2 * p
            visit(g0, 0, g0 + 2)
            visit(g0 + 1, 1, g0 + 3)
            return c

        lax.fori_loop(0, n_pairs - 1, pair, 0)

        g_last = n_groups - 2
        visit(g_last, 0, None)
        visit(g_last + 1, 1, None)
        drain_stores(g_last, 0)
        drain_stores(g_last + 1, 1)

    return k


def kernel(indices, W):
    batch, hist = indices.shape
    vocab, dim = W.shape
    total = batch * hist
    assert total % (_NW * _CHUNK) == 0 and dim % _LANES == 0
    n_chunks = total // (_NW * _CHUNK)

    idx = indices.reshape(_NW, n_chunks, _CHUNK).astype(jnp.int32)
    out = _make_gather(n_chunks, dim)(idx, W)
    return out.reshape(batch, hist, dim)


# speedup vs baseline: 1.4731x; 1.2256x over previous
import functools
import math

import jax
import jax.numpy as jnp
from jax import lax
from jax.experimental import pallas as pl
from jax.experimental.pallas import tpu as pltpu
from jax.experimental.pallas import tpu_sc as plsc

_INFO = plsc.get_sparse_core_info()
_NC = _INFO.num_cores
_NS = _INFO.num_subcores
_NW = _NC * _NS
_LANES = _INFO.num_lanes

_ROW_UNROLL = 8


def _segments(hist):
    segs, off = [], 0
    while off < hist:
        n = min(128, hist - off)
        segs.append((off, n))
        off += n
    assert all(o % 8 == 0 and n % 8 == 0 for o, n in segs)
    return segs


def _make_lookup(batch, hist, vocab, dim):
    mesh = plsc.VectorSubcoreMesh(core_axis_name="c", subcore_axis_name="s")
    rows_per_w = batch // _NW
    segs = _segments(hist)

    scratch = [pltpu.VMEM((rows_per_w, hist), jnp.int32)]
    for _ in range(2):
        scratch += [pltpu.VMEM((n, 2 * dim), jnp.float32) for _, n in segs]
    scratch += [pltpu.SemaphoreType.DMA for _ in range(4)]

    @functools.partial(
        pl.kernel,
        mesh=mesh,
        out_type=jax.ShapeDtypeStruct((batch, hist, 2 * dim), jnp.float32),
        scratch_types=scratch,
        compiler_params=pltpu.CompilerParams(use_tc_tiling_on_sc=True),
    )
    def k(idx_hbm, table_hbm, out_hbm, idx_v, *rest):
        ns = len(segs)
        bufs = (rest[:ns], rest[ns:2 * ns])
        sem_g = rest[2 * ns:2 * ns + 2]
        sem_s = rest[2 * ns + 2:2 * ns + 4]

        wid = lax.axis_index("s") * _NC + lax.axis_index("c")
        row0 = wid * rows_per_w
        pltpu.sync_copy(idx_hbm.at[pl.ds(row0, rows_per_w)], idx_v)

        def issue_gathers(r, bank):
            for j, (off, n) in enumerate(segs):
                pltpu.async_copy(
                    table_hbm.at[idx_v.at[r, pl.ds(off, n)]],
                    bufs[bank][j], sem_g[bank])

        def drain_gathers(r, bank):
            for j, (off, n) in enumerate(segs):
                pltpu.make_async_copy(
                    table_hbm.at[idx_v.at[r, pl.ds(off, n)]],
                    bufs[bank][j], sem_g[bank]).wait()

        def scale(bank, j, n):
            buf = bufs[bank][j]

            def body(i, c):
                base = i * _ROW_UNROLL
                for r in range(_ROW_UNROLL):
                    for col in range(dim // _LANES):
                        sl = pl.ds(col * _LANES, _LANES)
                        buf[base + r, sl] = buf[base + r, sl] * 8.0
                return c

            lax.fori_loop(0, n // _ROW_UNROLL, body, 0)

        def issue_stores(r, bank):
            for j, (off, n) in enumerate(segs):
                scale(bank, j, n)
                pltpu.async_copy(
                    bufs[bank][j],
                    out_hbm.at[row0 + r, pl.ds(off, n)], sem_s[bank])

        def drain_stores(r, bank):
            for j, (off, n) in enumerate(segs):
                pltpu.make_async_copy(
                    bufs[bank][j],
                    out_hbm.at[row0 + r, pl.ds(off, n)], sem_s[bank]).wait()

        def visit(r, bank, reissue):
            drain_gathers(r, bank)
            issue_stores(r, bank)
            drain_stores(r, bank)
            if reissue:
                issue_gathers(r + 2, bank)

        issue_gathers(0, 0)
        issue_gathers(1, 1)

        def pair(p, c):
            visit(2 * p, 0, True)
            visit(2 * p + 1, 1, True)
            return c

        lax.fori_loop(0, rows_per_w // 2 - 1, pair, 0)

        r_last = rows_per_w - 2
        visit(r_last, 0, False)
        visit(r_last + 1, 1, False)

    return k


def kernel(indices, W):
    batch, hist = indices.shape
    vocab, dim = W.shape
    assert batch % _NW == 0 and dim % _LANES == 0 and 2 * dim == 128
    idx = indices if indices.dtype == jnp.int32 else indices.astype(jnp.int32)
    table = jnp.pad(W, ((0, 0), (0, dim)))
    out = _make_lookup(batch, hist, vocab, dim)(idx, table)
    return out[:, :, :dim]
